# baseline (speedup 1.0000x reference)
import functools

import jax
import jax.numpy as jnp
from jax import lax
from jax.experimental import pallas as pl
from jax.experimental.pallas import tpu as pltpu
from jax.experimental.pallas import tpu_sc as plsc

_CHUNK = 32
_SB = 4
_CHUNK2 = 64
_LANES = 16
_CB = 512


def _sc_species(species_idx, spT):
    B = species_idx.shape[0]
    info = plsc.get_sparse_core_info()
    nw = info.num_cores * info.num_subcores
    per_w = B // nw
    nchunk = per_w // _CHUNK
    nsb = _CHUNK // _SB

    mesh = plsc.VectorSubcoreMesh(core_axis_name="c", subcore_axis_name="s")

    @functools.partial(
        pl.kernel,
        out_type=jax.ShapeDtypeStruct((B, 128), jnp.float32),
        mesh=mesh,
        compiler_params=pltpu.CompilerParams(needs_layout_passes=False),
        scratch_types=[
            pltpu.VMEM((per_w,), jnp.int32),
            *[pltpu.VMEM((_SB, 64, 128), jnp.float32) for _ in range(2)],
            pltpu.VMEM((_CHUNK, 128), jnp.float32),
            pltpu.SemaphoreType.DMA,
            pltpu.SemaphoreType.DMA,
        ],
    )
    def k(sp_hbm, tsp, out_hbm, sv, st0, st1, ob, semA, semB):
        cid = lax.axis_index("c")
        sid = lax.axis_index("s")
        wid = sid * info.num_cores + cid
        base = wid * per_w
        pltpu.sync_copy(sp_hbm.at[pl.ds(base, per_w)], sv)

        zero = jnp.zeros((_LANES,), jnp.float32)

        def zbody(i, carry):
            for s in range(4):
                ob[i, pl.ds(64 + s * _LANES, _LANES)] = zero
            return carry

        lax.fori_loop(0, _CHUNK, zbody, 0)

        iota = lax.iota(jnp.int32, _LANES)
        stages = (st0, st1)
        sems = (semA, semB)

        def fire_sb(sb, cb, buf):
            g, l0 = divmod(sb * _SB, _LANES)
            rv = sv[pl.ds(cb + g * _LANES, _LANES)]
            ops = []
            for q in range(_SB):
                r = rv[l0 + q]
                blk = pl.multiple_of(
                    lax.shift_left(lax.shift_right_logical(r, 7), 7), 128)
                ops.append(pltpu.async_copy(
                    tsp.at[:, pl.ds(blk, 128)], stages[buf].at[q],
                    sems[buf]))
            return ops

        def extract_sb(sb, cb, buf):
            g, l0 = divmod(sb * _SB, _LANES)
            rv = sv[pl.ds(cb + g * _LANES, _LANES)]
            for q in range(_SB):
                i = sb * _SB + q
                cl = jnp.broadcast_to(rv[l0 + q] & 127, (_LANES,))
                for s in range(4):
                    fidx = iota + (s * _LANES)
                    qv = jnp.full((_LANES,), q, jnp.int32)
                    v = plsc.load_gather(stages[buf], [qv, fidx, cl])
                    ob[i, pl.ds(s * _LANES, _LANES)] = v

        def chunk_body(c, carry):
            cb = c * _CHUNK
            ops0 = fire_sb(0, cb, 0)
            for sb in range(nsb):
                cur = sb % 2
                if sb == 0:
                    cops = ops0
                for cop in cops:
                    cop.wait()
                if sb + 1 < nsb:
                    nops = fire_sb(sb + 1, cb, 1 - cur)
                extract_sb(sb, cb, cur)
                if sb + 1 < nsb:
                    cops = nops
            pltpu.sync_copy(ob, out_hbm.at[pl.ds(base + cb, _CHUNK)])
            return carry

        lax.fori_loop(0, nchunk, chunk_body, 0)

    return k(species_idx, spT)


def _tcT_body(in_ref, o_ref):
    t = jnp.transpose(in_ref[...], (1, 0))
    o_ref[...] = jnp.concatenate(
        [t, jnp.zeros((_CB, 64), jnp.float32)], axis=1)


def _tc_pad_transpose(tabT):
    N = tabT.shape[1]
    return pl.pallas_call(
        _tcT_body,
        grid=(pl.cdiv(N, _CB),),
        in_specs=[pl.BlockSpec((64, _CB), lambda i: (0, i))],
        out_specs=pl.BlockSpec((_CB, 128), lambda i: (i, 0)),
        out_shape=jax.ShapeDtypeStruct((N, 128), jnp.float32),
    )(tabT)


def _sc_smalls(ability_idx, item_idx, move_flat, abp, itp, acp):
    B = ability_idx.shape[0]
    info = plsc.get_sparse_core_info()
    nw = info.num_cores * info.num_subcores
    per_w = B // nw
    nchunk = per_w // _CHUNK2

    mesh = plsc.VectorSubcoreMesh(core_axis_name="c", subcore_axis_name="s")

    @functools.partial(
        pl.kernel,
        out_type=jax.ShapeDtypeStruct((B, 128), jnp.float32),
        mesh=mesh,
        compiler_params=pltpu.CompilerParams(needs_layout_passes=False),
        scratch_types=[
            pltpu.VMEM((per_w,), jnp.int32),
            pltpu.VMEM((per_w,), jnp.int32),
            pltpu.VMEM((4 * per_w,), jnp.int32),
            *[pltpu.VMEM((_CHUNK2, 128), jnp.float32) for _ in range(6)],
            pltpu.VMEM((_CHUNK2, 128), jnp.float32),
            pltpu.SemaphoreType.DMA,
        ],
    )
    def k(ab_hbm, it_hbm, mv_hbm, tab, tit, tac, out_hbm,
          av, iv, mv,
          r0, r1, r2, r3, r4, r5,
          ob, sem):
        cid = lax.axis_index("c")
        sid = lax.axis_index("s")
        wid = sid * info.num_cores + cid
        base = wid * per_w
        pltpu.sync_copy(ab_hbm.at[pl.ds(base, per_w)], av)
        pltpu.sync_copy(it_hbm.at[pl.ds(base, per_w)], iv)
        for j in range(4):
            pltpu.sync_copy(mv_hbm.at[pl.ds(j * B + base, per_w)],
                            mv.at[pl.ds(j * per_w, per_w)])

        zero = jnp.zeros((_LANES,), jnp.float32)

        def zbody(i, carry):
            for s in range(4):
                ob[i, pl.ds(64 + s * _LANES, _LANES)] = zero
            return carry

        lax.fori_loop(0, _CHUNK2, zbody, 0)

        rbufs = (r0, r1, r2, r3, r4, r5)

        def chunk_body(c, carry):
            cb = c * _CHUNK2
            idxs = (av.at[pl.ds(cb, _CHUNK2)],
                    iv.at[pl.ds(cb, _CHUNK2)],
                    mv.at[pl.ds(0 * per_w + cb, _CHUNK2)],
                    mv.at[pl.ds(1 * per_w + cb, _CHUNK2)],
                    mv.at[pl.ds(2 * per_w + cb, _CHUNK2)],
                    mv.at[pl.ds(3 * per_w + cb, _CHUNK2)])
            tbls = (tab, tit, tac, tac, tac, tac)
            mops = [pltpu.async_copy(tbls[k_].at[idxs[k_]], rbufs[k_], sem)
                    for k_ in range(6)]
            for mop in mops:
                mop.wait()

            def row_body(i, rcarry):
                for s in range(4):
                    sl = pl.ds(s * _LANES, _LANES)
                    v = r0[i, sl]
                    for k_ in range(1, 6):
                        v = v + rbufs[k_][i, sl]
                    ob[i, sl] = v
                return rcarry

            lax.fori_loop(0, _CHUNK2, row_body, 0)
            pltpu.sync_copy(ob, out_hbm.at[pl.ds(base + cb, _CHUNK2)])
            return carry

        lax.fori_loop(0, nchunk, chunk_body, 0)

    return k(ability_idx, item_idx, move_flat, abp, itp, acp)


def _mlp_body(sp_ref, sm_ref, w_ref, b_ref, s_ref, o_ref):
    emb = sp_ref[...] + sm_ref[...]
    h = jnp.dot(emb, w_ref[...], preferred_element_type=jnp.float32)
    h = jnp.maximum(h + b_ref[...], 0.0)
    mask = s_ref[...] != 0
    o_ref[...] = jnp.where(mask, h, 0.0)


def _tc_mlp(emb_sp, emb_sm, W, b, species_idx):
    B = emb_sp.shape[0]
    D = W.shape[0]
    blk = 2048
    wpad = jnp.concatenate([W, jnp.zeros((64, D), W.dtype)], axis=0)
    return pl.pallas_call(
        _mlp_body,
        grid=(B // blk,),
        in_specs=[
            pl.BlockSpec((blk, 128), lambda i: (i, 0)),
            pl.BlockSpec((blk, 128), lambda i: (i, 0)),
            pl.BlockSpec((128, D), lambda i: (0, 0)),
            pl.BlockSpec((1, D), lambda i: (0, 0)),
            pl.BlockSpec((blk, 1), lambda i: (i, 0)),
        ],
        out_specs=pl.BlockSpec((blk, D), lambda i: (i, 0)),
        out_shape=jax.ShapeDtypeStruct((B, D), jnp.float32),
    )(emb_sp, emb_sm, wpad, b.reshape(1, D), species_idx.reshape(B, 1))


def kernel(species_idx, ability_idx, item_idx, move_idx,
           species_table, ability_table, item_table, action_table, W, b):
    move_flat = move_idx.T.reshape(-1)
    spT = species_table.T
    emb_sp = _sc_species(species_idx, spT)
    abp = _tc_pad_transpose(ability_table.T)
    itp = _tc_pad_transpose(item_table.T)
    acp = _tc_pad_transpose(action_table.T)
    emb_sm = _sc_smalls(ability_idx, item_idx, move_flat, abp, itp, acp)
    return _tc_mlp(emb_sp, emb_sm, W, b, species_idx)


# speedup vs baseline: 1.6016x; 1.6016x over previous
import functools

import jax
import jax.numpy as jnp
from jax import lax
from jax.experimental import pallas as pl
from jax.experimental.pallas import tpu as pltpu
from jax.experimental.pallas import tpu_sc as plsc

_CHUNK = 32
_SB = 4
_LANES = 16


def _sc_species(species_idx, spT):
    B = species_idx.shape[0]
    info = plsc.get_sparse_core_info()
    nw = info.num_cores * info.num_subcores
    per_w = B // nw
    nchunk = per_w // _CHUNK
    nsb = _CHUNK // _SB

    mesh = plsc.VectorSubcoreMesh(core_axis_name="c", subcore_axis_name="s")

    @functools.partial(
        pl.kernel,
        out_type=jax.ShapeDtypeStruct((B, 128), jnp.float32),
        mesh=mesh,
        compiler_params=pltpu.CompilerParams(needs_layout_passes=False),
        scratch_types=[
            pltpu.VMEM((per_w,), jnp.int32),
            *[pltpu.VMEM((_SB, 64, 128), jnp.float32) for _ in range(3)],
            pltpu.VMEM((_CHUNK, 128), jnp.float32),
            pltpu.SemaphoreType.DMA,
            pltpu.SemaphoreType.DMA,
            pltpu.SemaphoreType.DMA,
        ],
    )
    def k(sp_hbm, tsp, out_hbm, sv, st0, st1, st2, ob, semA, semB, semC):
        cid = lax.axis_index("c")
        sid = lax.axis_index("s")
        wid = sid * info.num_cores + cid
        base = wid * per_w
        pltpu.sync_copy(sp_hbm.at[pl.ds(base, per_w)], sv)

        zero = jnp.zeros((_LANES,), jnp.float32)

        def zbody(i, carry):
            for s in range(4):
                ob[i, pl.ds(64 + s * _LANES, _LANES)] = zero
            return carry

        lax.fori_loop(0, _CHUNK, zbody, 0)

        iota = lax.iota(jnp.int32, _LANES)
        stages = (st0, st1, st2)
        sems = (semA, semB, semC)

        def fire_sb(sb, cb, buf):
            g, l0 = divmod(sb * _SB, _LANES)
            rv = sv[pl.ds(cb + g * _LANES, _LANES)]
            ops = []
            for q in range(_SB):
                r = rv[l0 + q]
                blk = pl.multiple_of(
                    lax.shift_left(lax.shift_right_logical(r, 7), 7), 128)
                ops.append(pltpu.async_copy(
                    tsp.at[:, pl.ds(blk, 128)], stages[buf].at[q],
                    sems[buf]))
            return ops

        def extract_sb(sb, cb, buf):
            g, l0 = divmod(sb * _SB, _LANES)
            rv = sv[pl.ds(cb + g * _LANES, _LANES)]
            for q in range(_SB):
                i = sb * _SB + q
                cl = jnp.broadcast_to(rv[l0 + q] & 127, (_LANES,))
                for s in range(4):
                    fidx = iota + (s * _LANES)
                    qv = jnp.full((_LANES,), q, jnp.int32)
                    v = plsc.load_gather(stages[buf], [qv, fidx, cl])
                    ob[i, pl.ds(s * _LANES, _LANES)] = v

        def chunk_body(c, carry):
            cb = c * _CHUNK
            pend = [fire_sb(0, cb, 0), fire_sb(1, cb, 1)]
            for sb in range(nsb):
                cur = sb % 3
                for cop in pend.pop(0):
                    cop.wait()
                if sb + 2 < nsb:
                    pend.append(fire_sb(sb + 2, cb, (sb + 2) % 3))
                extract_sb(sb, cb, cur)
            pltpu.sync_copy(ob, out_hbm.at[pl.ds(base + cb, _CHUNK)])
            return carry

        lax.fori_loop(0, nchunk, chunk_body, 0)

    return k(species_idx, spT)


def _sc_smalls(ability_idx, item_idx, move_flat, abt2, itt2, act2):
    B = ability_idx.shape[0]
    info = plsc.get_sparse_core_info()
    nw = info.num_cores * info.num_subcores
    per_w = B // nw
    nchunk = per_w // _CHUNK

    mesh = plsc.VectorSubcoreMesh(core_axis_name="c", subcore_axis_name="s")

    @functools.partial(
        pl.kernel,
        out_type=jax.ShapeDtypeStruct((B, 128), jnp.float32),
        mesh=mesh,
        compiler_params=pltpu.CompilerParams(needs_layout_passes=False),
        scratch_types=[
            pltpu.VMEM((per_w,), jnp.int32),
            pltpu.VMEM((per_w,), jnp.int32),
            pltpu.VMEM((4 * per_w,), jnp.int32),
            *[pltpu.VMEM((_CHUNK,), jnp.int32) for _ in range(6)],
            *[pltpu.VMEM((_CHUNK, 128), jnp.float32) for _ in range(6)],
            pltpu.VMEM((_CHUNK, 128), jnp.float32),
            pltpu.SemaphoreType.DMA,
        ],
    )
    def k(ab_hbm, it_hbm, mv_hbm, tab, tit, tac, out_hbm,
          av, iv, mv,
          g0, g1, g2, g3, g4, g5,
          r0, r1, r2, r3, r4, r5,
          ob, sem):
        cid = lax.axis_index("c")
        sid = lax.axis_index("s")
        wid = sid * info.num_cores + cid
        base = wid * per_w
        pltpu.sync_copy(ab_hbm.at[pl.ds(base, per_w)], av)
        pltpu.sync_copy(it_hbm.at[pl.ds(base, per_w)], iv)
        for j in range(4):
            pltpu.sync_copy(mv_hbm.at[pl.ds(j * B + base, per_w)],
                            mv.at[pl.ds(j * per_w, per_w)])

        zero = jnp.zeros((_LANES,), jnp.float32)

        def zbody(i, carry):
            for s in range(4):
                ob[i, pl.ds(64 + s * _LANES, _LANES)] = zero
            return carry

        lax.fori_loop(0, _CHUNK, zbody, 0)

        gbufs = (g0, g1, g2, g3, g4, g5)
        rbufs = (r0, r1, r2, r3, r4, r5)
        tbls = (tab, tit, tac, tac, tac, tac)

        def src_off(k_, cb_):
            if k_ < 2:
                return cb_
            return (k_ - 2) * per_w + cb_

        ivecs = (av, iv, mv, mv, mv, mv)

        def chunk_body(c, carry):
            cb = c * _CHUNK
            for k_ in range(6):
                off = src_off(k_, cb)
                for g in range(_CHUNK // _LANES):
                    v = ivecs[k_][pl.ds(off + g * _LANES, _LANES)]
                    gbufs[k_][pl.ds(g * _LANES, _LANES)] = (
                        lax.shift_right_logical(v, 1))
            mops = [pltpu.async_copy(tbls[k_].at[gbufs[k_]], rbufs[k_], sem)
                    for k_ in range(6)]
            for mop in mops:
                mop.wait()

            def group_body(g, gcarry):
                hvecs = [(ivecs[k_][pl.ds(src_off(k_, cb) + g * _LANES,
                                          _LANES)] & 1) * 64
                         for k_ in range(6)]
                for l in range(_LANES):
                    i = g * _LANES + l
                    offs = [hv[l] for hv in hvecs]
                    for s in range(4):
                        v = r0[i, pl.ds(offs[0] + s * _LANES, _LANES)]
                        for k_ in range(1, 6):
                            v = v + rbufs[k_][i, pl.ds(offs[k_] + s * _LANES,
                                                       _LANES)]
                        ob[i, pl.ds(s * _LANES, _LANES)] = v
                return gcarry

            lax.fori_loop(0, _CHUNK // _LANES, group_body, 0)
            pltpu.sync_copy(ob, out_hbm.at[pl.ds(base + cb, _CHUNK)])
            return carry

        lax.fori_loop(0, nchunk, chunk_body, 0)

    return k(ability_idx, item_idx, move_flat, abt2, itt2, act2)


def _mlp_body(sp_ref, sm_ref, w_ref, b_ref, s_ref, o_ref):
    emb = sp_ref[...] + sm_ref[...]
    h = jnp.dot(emb, w_ref[...], preferred_element_type=jnp.float32)
    h = jnp.maximum(h + b_ref[...], 0.0)
    mask = s_ref[...] != 0
    o_ref[...] = jnp.where(mask, h, 0.0)


def _tc_mlp(emb_sp, emb_sm, W, b, species_idx):
    B = emb_sp.shape[0]
    D = W.shape[0]
    blk = 2048
    wpad = jnp.concatenate([W, jnp.zeros((64, D), W.dtype)], axis=0)
    return pl.pallas_call(
        _mlp_body,
        grid=(B // blk,),
        in_specs=[
            pl.BlockSpec((blk, 128), lambda i: (i, 0)),
            pl.BlockSpec((blk, 128), lambda i: (i, 0)),
            pl.BlockSpec((128, D), lambda i: (0, 0)),
            pl.BlockSpec((1, D), lambda i: (0, 0)),
            pl.BlockSpec((blk, 1), lambda i: (i, 0)),
        ],
        out_specs=pl.BlockSpec((blk, D), lambda i: (i, 0)),
        out_shape=jax.ShapeDtypeStruct((B, D), jnp.float32),
    )(emb_sp, emb_sm, wpad, b.reshape(1, D), species_idx.reshape(B, 1))


def kernel(species_idx, ability_idx, item_idx, move_idx,
           species_table, ability_table, item_table, action_table, W, b):
    move_flat = move_idx.T.reshape(-1)
    spT = species_table.T
    emb_sp = _sc_species(species_idx, spT)
    abt2 = ability_table.reshape(-1, 128)
    itt2 = item_table.reshape(-1, 128)
    act2 = action_table.reshape(-1, 128)
    emb_sm = _sc_smalls(ability_idx, item_idx, move_flat, abt2, itt2, act2)
    return _tc_mlp(emb_sp, emb_sm, W, b, species_idx)


# speedup vs baseline: 1.6283x; 1.0167x over previous
import functools

import jax
import jax.numpy as jnp
from jax import lax
from jax.experimental import pallas as pl
from jax.experimental.pallas import tpu as pltpu
from jax.experimental.pallas import tpu_sc as plsc

_CHUNK = 64
_SB = 4
_LANES = 16


def _sc_species(species_idx, spT):
    B = species_idx.shape[0]
    info = plsc.get_sparse_core_info()
    nw = info.num_cores * info.num_subcores
    per_w = B // nw
    nchunk = per_w // _CHUNK
    nsb = _CHUNK // _SB

    mesh = plsc.VectorSubcoreMesh(core_axis_name="c", subcore_axis_name="s")

    @functools.partial(
        pl.kernel,
        out_type=jax.ShapeDtypeStruct((B, 128), jnp.float32),
        mesh=mesh,
        compiler_params=pltpu.CompilerParams(needs_layout_passes=False),
        scratch_types=[
            pltpu.VMEM((per_w,), jnp.int32),
            *[pltpu.VMEM((_SB, 64, 128), jnp.float32) for _ in range(3)],
            pltpu.VMEM((_CHUNK, 128), jnp.float32),
            pltpu.SemaphoreType.DMA,
            pltpu.SemaphoreType.DMA,
            pltpu.SemaphoreType.DMA,
        ],
    )
    def k(sp_hbm, tsp, out_hbm, sv, st0, st1, st2, ob, semA, semB, semC):
        cid = lax.axis_index("c")
        sid = lax.axis_index("s")
        wid = sid * info.num_cores + cid
        base = wid * per_w
        pltpu.sync_copy(sp_hbm.at[pl.ds(base, per_w)], sv)

        zero = jnp.zeros((_LANES,), jnp.float32)

        def zbody(i, carry):
            for s in range(4):
                ob[i, pl.ds(64 + s * _LANES, _LANES)] = zero
            return carry

        lax.fori_loop(0, _CHUNK, zbody, 0)

        iota = lax.iota(jnp.int32, _LANES)
        stages = (st0, st1, st2)
        sems = (semA, semB, semC)

        def fire_sb(sb, cb, buf):
            g, l0 = divmod(sb * _SB, _LANES)
            rv = sv[pl.ds(cb + g * _LANES, _LANES)]
            ops = []
            for q in range(_SB):
                r = rv[l0 + q]
                blk = pl.multiple_of(
                    lax.shift_left(lax.shift_right_logical(r, 7), 7), 128)
                ops.append(pltpu.async_copy(
                    tsp.at[:, pl.ds(blk, 128)], stages[buf].at[q],
                    sems[buf]))
            return ops

        def extract_sb(sb, cb, buf):
            g, l0 = divmod(sb * _SB, _LANES)
            rv = sv[pl.ds(cb + g * _LANES, _LANES)]
            for q in range(_SB):
                i = sb * _SB + q
                cl = jnp.broadcast_to(rv[l0 + q] & 127, (_LANES,))
                for s in range(4):
                    fidx = iota + (s * _LANES)
                    qv = jnp.full((_LANES,), q, jnp.int32)
                    v = plsc.load_gather(stages[buf], [qv, fidx, cl])
                    ob[i, pl.ds(s * _LANES, _LANES)] = v

        def chunk_body(c, carry):
            cb = c * _CHUNK
            pend = [fire_sb(0, cb, 0), fire_sb(1, cb, 1)]
            for sb in range(nsb):
                cur = sb % 3
                for cop in pend.pop(0):
                    cop.wait()
                if sb + 2 < nsb:
                    pend.append(fire_sb(sb + 2, cb, (sb + 2) % 3))
                extract_sb(sb, cb, cur)
            pltpu.sync_copy(ob, out_hbm.at[pl.ds(base + cb, _CHUNK)])
            return carry

        lax.fori_loop(0, nchunk, chunk_body, 0)

    return k(species_idx, spT)


def _sc_smalls(ability_idx, item_idx, move_flat, abt2, itt2, act2):
    B = ability_idx.shape[0]
    info = plsc.get_sparse_core_info()
    nw = info.num_cores * info.num_subcores
    per_w = B // nw
    nchunk = per_w // _CHUNK

    mesh = plsc.VectorSubcoreMesh(core_axis_name="c", subcore_axis_name="s")

    @functools.partial(
        pl.kernel,
        out_type=jax.ShapeDtypeStruct((B, 128), jnp.float32),
        mesh=mesh,
        compiler_params=pltpu.CompilerParams(needs_layout_passes=False),
        scratch_types=[
            pltpu.VMEM((per_w,), jnp.int32),
            pltpu.VMEM((per_w,), jnp.int32),
            pltpu.VMEM((4 * per_w,), jnp.int32),
            *[pltpu.VMEM((_CHUNK,), jnp.int32) for _ in range(6)],
            *[pltpu.VMEM((_CHUNK, 128), jnp.float32) for _ in range(6)],
            pltpu.VMEM((_CHUNK, 128), jnp.float32),
            pltpu.SemaphoreType.DMA,
        ],
    )
    def k(ab_hbm, it_hbm, mv_hbm, tab, tit, tac, out_hbm,
          av, iv, mv,
          g0, g1, g2, g3, g4, g5,
          r0, r1, r2, r3, r4, r5,
          ob, sem):
        cid = lax.axis_index("c")
        sid = lax.axis_index("s")
        wid = sid * info.num_cores + cid
        base = wid * per_w
        pltpu.sync_copy(ab_hbm.at[pl.ds(base, per_w)], av)
        pltpu.sync_copy(it_hbm.at[pl.ds(base, per_w)], iv)
        for j in range(4):
            pltpu.sync_copy(mv_hbm.at[pl.ds(j * B + base, per_w)],
                            mv.at[pl.ds(j * per_w, per_w)])

        zero = jnp.zeros((_LANES,), jnp.float32)

        def zbody(i, carry):
            for s in range(4):
                ob[i, pl.ds(64 + s * _LANES, _LANES)] = zero
            return carry

        lax.fori_loop(0, _CHUNK, zbody, 0)

        gbufs = (g0, g1, g2, g3, g4, g5)
        rbufs = (r0, r1, r2, r3, r4, r5)
        tbls = (tab, tit, tac, tac, tac, tac)

        def src_off(k_, cb_):
            if k_ < 2:
                return cb_
            return (k_ - 2) * per_w + cb_

        ivecs = (av, iv, mv, mv, mv, mv)

        def chunk_body(c, carry):
            cb = c * _CHUNK
            for k_ in range(6):
                off = src_off(k_, cb)
                for g in range(_CHUNK // _LANES):
                    v = ivecs[k_][pl.ds(off + g * _LANES, _LANES)]
                    gbufs[k_][pl.ds(g * _LANES, _LANES)] = (
                        lax.shift_right_logical(v, 1))
            mops = [pltpu.async_copy(tbls[k_].at[gbufs[k_]], rbufs[k_], sem)
                    for k_ in range(6)]
            for mop in mops:
                mop.wait()

            def group_body(g, gcarry):
                hvecs = [(ivecs[k_][pl.ds(src_off(k_, cb) + g * _LANES,
                                          _LANES)] & 1) * 64
                         for k_ in range(6)]
                for l in range(_LANES):
                    i = g * _LANES + l
                    offs = [hv[l] for hv in hvecs]
                    for s in range(4):
                        v = r0[i, pl.ds(offs[0] + s * _LANES, _LANES)]
                        for k_ in range(1, 6):
                            v = v + rbufs[k_][i, pl.ds(offs[k_] + s * _LANES,
                                                       _LANES)]
                        ob[i, pl.ds(s * _LANES, _LANES)] = v
                return gcarry

            lax.fori_loop(0, _CHUNK // _LANES, group_body, 0)
            pltpu.sync_copy(ob, out_hbm.at[pl.ds(base + cb, _CHUNK)])
            return carry

        lax.fori_loop(0, nchunk, chunk_body, 0)

    return k(ability_idx, item_idx, move_flat, abt2, itt2, act2)


def _mlp_body(sp_ref, sm_ref, w_ref, b_ref, s_ref, o_ref):
    emb = sp_ref[...] + sm_ref[...]
    h = jnp.dot(emb, w_ref[...], preferred_element_type=jnp.float32)
    h = jnp.maximum(h + b_ref[...], 0.0)
    mask = s_ref[...] != 0
    o_ref[...] = jnp.where(mask, h, 0.0)


def _tc_mlp(emb_sp, emb_sm, W, b, species_idx):
    B = emb_sp.shape[0]
    D = W.shape[0]
    blk = 2048
    wpad = jnp.concatenate([W, jnp.zeros((64, D), W.dtype)], axis=0)
    return pl.pallas_call(
        _mlp_body,
        grid=(B // blk,),
        in_specs=[
            pl.BlockSpec((blk, 128), lambda i: (i, 0)),
            pl.BlockSpec((blk, 128), lambda i: (i, 0)),
            pl.BlockSpec((128, D), lambda i: (0, 0)),
            pl.BlockSpec((1, D), lambda i: (0, 0)),
            pl.BlockSpec((blk, 1), lambda i: (i, 0)),
        ],
        out_specs=pl.BlockSpec((blk, D), lambda i: (i, 0)),
        out_shape=jax.ShapeDtypeStruct((B, D), jnp.float32),
    )(emb_sp, emb_sm, wpad, b.reshape(1, D), species_idx.reshape(B, 1))


def kernel(species_idx, ability_idx, item_idx, move_idx,
           species_table, ability_table, item_table, action_table, W, b):
    move_flat = move_idx.T.reshape(-1)
    spT = species_table.T
    emb_sp = _sc_species(species_idx, spT)
    abt2 = ability_table.reshape(-1, 128)
    itt2 = item_table.reshape(-1, 128)
    act2 = action_table.reshape(-1, 128)
    emb_sm = _sc_smalls(ability_idx, item_idx, move_flat, abt2, itt2, act2)
    return _tc_mlp(emb_sp, emb_sm, W, b, species_idx)


# speedup vs baseline: 1.6519x; 1.0145x over previous
import functools

import jax
import jax.numpy as jnp
from jax import lax
from jax.experimental import pallas as pl
from jax.experimental.pallas import tpu as pltpu
from jax.experimental.pallas import tpu_sc as plsc

_CHUNK = 64
_SB = 4
_LANES = 16


def _sc_species(species_idx, spT):
    B = species_idx.shape[0]
    info = plsc.get_sparse_core_info()
    nw = info.num_cores * info.num_subcores
    per_w = B // nw
    nchunk = per_w // _CHUNK
    nsb = _CHUNK // _SB

    mesh = plsc.VectorSubcoreMesh(core_axis_name="c", subcore_axis_name="s")

    @functools.partial(
        pl.kernel,
        out_type=jax.ShapeDtypeStruct((B, 128), jnp.float32),
        mesh=mesh,
        compiler_params=pltpu.CompilerParams(needs_layout_passes=False),
        scratch_types=[
            pltpu.VMEM((per_w,), jnp.int32),
            *[pltpu.VMEM((_SB, 64, 128), jnp.float32) for _ in range(3)],
            pltpu.VMEM((_CHUNK, 128), jnp.float32),
            pltpu.SemaphoreType.DMA,
            pltpu.SemaphoreType.DMA,
            pltpu.SemaphoreType.DMA,
        ],
    )
    def k(sp_hbm, tsp, out_hbm, sv, st0, st1, st2, ob, semA, semB, semC):
        cid = lax.axis_index("c")
        sid = lax.axis_index("s")
        wid = sid * info.num_cores + cid
        base = wid * per_w
        pltpu.sync_copy(sp_hbm.at[pl.ds(base, per_w)], sv)

        zero = jnp.zeros((_LANES,), jnp.float32)

        def zbody(i, carry):
            for s in range(4):
                ob[i, pl.ds(64 + s * _LANES, _LANES)] = zero
            return carry

        lax.fori_loop(0, _CHUNK, zbody, 0)

        iota = lax.iota(jnp.int32, _LANES)
        stages = (st0, st1, st2)
        sems = (semA, semB, semC)

        def fire_sb(sb, cb, buf):
            g, l0 = divmod(sb * _SB, _LANES)
            rv = sv[pl.ds(cb + g * _LANES, _LANES)]
            ops = []
            for q in range(_SB):
                r = rv[l0 + q]
                blk = pl.multiple_of(
                    lax.shift_left(lax.shift_right_logical(r, 7), 7), 128)
                ops.append(pltpu.async_copy(
                    tsp.at[:, pl.ds(blk, 128)], stages[buf].at[q],
                    sems[buf]))
            return ops

        def extract_sb(sb, cb, buf):
            g, l0 = divmod(sb * _SB, _LANES)
            rv = sv[pl.ds(cb + g * _LANES, _LANES)]
            for q in range(_SB):
                i = sb * _SB + q
                cl = jnp.broadcast_to(rv[l0 + q] & 127, (_LANES,))
                for s in range(4):
                    fidx = iota + (s * _LANES)
                    qv = jnp.full((_LANES,), q, jnp.int32)
                    v = plsc.load_gather(stages[buf], [qv, fidx, cl])
                    ob[i, pl.ds(s * _LANES, _LANES)] = v

        def chunk_body(c, carry):
            cb = c * _CHUNK
            pend = [fire_sb(0, cb, 0), fire_sb(1, cb, 1)]
            for sb in range(nsb):
                cur = sb % 3
                for cop in pend.pop(0):
                    cop.wait()
                if sb + 2 < nsb:
                    pend.append(fire_sb(sb + 2, cb, (sb + 2) % 3))
                extract_sb(sb, cb, cur)
            pltpu.sync_copy(ob, out_hbm.at[pl.ds(base + cb, _CHUNK)])
            return carry

        lax.fori_loop(0, nchunk, chunk_body, 0)

    return k(species_idx, spT)


def _sc_smalls(ability_idx, item_idx, move_flat, abt2, itt2, act2):
    B = ability_idx.shape[0]
    info = plsc.get_sparse_core_info()
    nw = info.num_cores * info.num_subcores
    per_w = B // nw
    nchunk = per_w // _CHUNK

    mesh = plsc.VectorSubcoreMesh(core_axis_name="c", subcore_axis_name="s")

    @functools.partial(
        pl.kernel,
        out_type=jax.ShapeDtypeStruct((B, 128), jnp.float32),
        mesh=mesh,
        compiler_params=pltpu.CompilerParams(needs_layout_passes=False),
        scratch_types=[
            pltpu.VMEM((per_w,), jnp.int32),
            pltpu.VMEM((per_w,), jnp.int32),
            pltpu.VMEM((4 * per_w,), jnp.int32),
            *[pltpu.VMEM((_CHUNK,), jnp.int32) for _ in range(12)],
            *[pltpu.VMEM((_CHUNK, 128), jnp.float32) for _ in range(12)],
            pltpu.VMEM((_CHUNK, 128), jnp.float32),
            pltpu.SemaphoreType.DMA,
            pltpu.SemaphoreType.DMA,
        ],
    )
    def k(ab_hbm, it_hbm, mv_hbm, tab, tit, tac, out_hbm,
          av, iv, mv,
          g0, g1, g2, g3, g4, g5, g6, g7, g8, g9, g10, g11,
          r0, r1, r2, r3, r4, r5, r6, r7, r8, r9, r10, r11,
          ob, semA, semB):
        cid = lax.axis_index("c")
        sid = lax.axis_index("s")
        wid = sid * info.num_cores + cid
        base = wid * per_w
        pltpu.sync_copy(ab_hbm.at[pl.ds(base, per_w)], av)
        pltpu.sync_copy(it_hbm.at[pl.ds(base, per_w)], iv)
        for j in range(4):
            pltpu.sync_copy(mv_hbm.at[pl.ds(j * B + base, per_w)],
                            mv.at[pl.ds(j * per_w, per_w)])

        zero = jnp.zeros((_LANES,), jnp.float32)

        def zbody(i, carry):
            for s in range(4):
                ob[i, pl.ds(64 + s * _LANES, _LANES)] = zero
            return carry

        lax.fori_loop(0, _CHUNK, zbody, 0)

        gsets = ((g0, g1, g2, g3, g4, g5), (g6, g7, g8, g9, g10, g11))
        rsets = ((r0, r1, r2, r3, r4, r5), (r6, r7, r8, r9, r10, r11))
        sems = (semA, semB)
        tbls = (tab, tit, tac, tac, tac, tac)

        def src_off(k_, cb_):
            if k_ < 2:
                return cb_
            return (k_ - 2) * per_w + cb_

        ivecs = (av, iv, mv, mv, mv, mv)

        def fire(c, s):
            cb = c * _CHUNK
            for k_ in range(6):
                off = src_off(k_, cb)
                for g in range(_CHUNK // _LANES):
                    v = ivecs[k_][pl.ds(off + g * _LANES, _LANES)]
                    gsets[s][k_][pl.ds(g * _LANES, _LANES)] = (
                        lax.shift_right_logical(v, 1))
            return [pltpu.async_copy(tbls[k_].at[gsets[s][k_]],
                                     rsets[s][k_], sems[s])
                    for k_ in range(6)]

        pend = fire(0, 0)
        for c in range(nchunk):
            s = c % 2
            cb = c * _CHUNK
            if c + 1 < nchunk:
                nops = fire(c + 1, 1 - s)
            for mop in pend:
                mop.wait()
            if c + 1 < nchunk:
                pend = nops
            rbufs = rsets[s]

            def group_body(g, gcarry, cb=cb, rbufs=rbufs):
                hvecs = [(ivecs[k_][pl.ds(src_off(k_, cb) + g * _LANES,
                                          _LANES)] & 1) * 64
                         for k_ in range(6)]
                for l in range(_LANES):
                    i = g * _LANES + l
                    offs = [hv[l] for hv in hvecs]
                    for s_ in range(4):
                        v = rbufs[0][i, pl.ds(offs[0] + s_ * _LANES, _LANES)]
                        for k_ in range(1, 6):
                            v = v + rbufs[k_][i, pl.ds(offs[k_] + s_ * _LANES,
                                                       _LANES)]
                        ob[i, pl.ds(s_ * _LANES, _LANES)] = v
                return gcarry

            lax.fori_loop(0, _CHUNK // _LANES, group_body, 0)
            pltpu.sync_copy(ob, out_hbm.at[pl.ds(base + cb, _CHUNK)])

    return k(ability_idx, item_idx, move_flat, abt2, itt2, act2)


def _mlp_body(sp_ref, sm_ref, w_ref, b_ref, s_ref, o_ref):
    emb = sp_ref[...] + sm_ref[...]
    h = jnp.dot(emb, w_ref[...], preferred_element_type=jnp.float32)
    h = jnp.maximum(h + b_ref[...], 0.0)
    mask = s_ref[...] != 0
    o_ref[...] = jnp.where(mask, h, 0.0)


def _tc_mlp(emb_sp, emb_sm, W, b, species_idx):
    B = emb_sp.shape[0]
    D = W.shape[0]
    blk = 2048
    wpad = jnp.concatenate([W, jnp.zeros((64, D), W.dtype)], axis=0)
    return pl.pallas_call(
        _mlp_body,
        grid=(B // blk,),
        in_specs=[
            pl.BlockSpec((blk, 128), lambda i: (i, 0)),
            pl.BlockSpec((blk, 128), lambda i: (i, 0)),
            pl.BlockSpec((128, D), lambda i: (0, 0)),
            pl.BlockSpec((1, D), lambda i: (0, 0)),
            pl.BlockSpec((blk, 1), lambda i: (i, 0)),
        ],
        out_specs=pl.BlockSpec((blk, D), lambda i: (i, 0)),
        out_shape=jax.ShapeDtypeStruct((B, D), jnp.float32),
    )(emb_sp, emb_sm, wpad, b.reshape(1, D), species_idx.reshape(B, 1))


def kernel(species_idx, ability_idx, item_idx, move_idx,
           species_table, ability_table, item_table, action_table, W, b):
    move_flat = move_idx.T.reshape(-1)
    spT = species_table.T
    emb_sp = _sc_species(species_idx, spT)
    abt2 = ability_table.reshape(-1, 128)
    itt2 = item_table.reshape(-1, 128)
    act2 = action_table.reshape(-1, 128)
    emb_sm = _sc_smalls(ability_idx, item_idx, move_flat, abt2, itt2, act2)
    return _tc_mlp(emb_sp, emb_sm, W, b, species_idx)
